# baseline (speedup 1.0000x reference)
import functools

import jax
import jax.numpy as jnp
from jax import lax
from jax.experimental import pallas as pl
from jax.experimental.pallas import tpu as pltpu
from jax.experimental.pallas import tpu_sc as plsc

EPS = 1e-12
CHUNK = 128


def _sc_gather(table, idx2d):
    n_chunks, _ = idx2d.shape
    D = table.shape[1]
    info = plsc.get_sparse_core_info()
    NC, NS = info.num_cores, info.num_subcores
    NW = NC * NS
    cpw = n_chunks // NW
    mesh = plsc.VectorSubcoreMesh(core_axis_name="c", subcore_axis_name="s")

    @functools.partial(
        pl.kernel,
        mesh=mesh,
        out_type=jax.ShapeDtypeStruct((n_chunks * CHUNK, D), jnp.float32),
        scratch_types=[
            pltpu.VMEM((CHUNK,), jnp.int32),
            pltpu.VMEM((CHUNK, D), jnp.float32),
            pltpu.SemaphoreType.DMA,
        ],
        compiler_params=pltpu.CompilerParams(use_tc_tiling_on_sc=False),
    )
    def k(table_hbm, idx_hbm, out_hbm, idx_v, rows_v, sem):
        wid = lax.axis_index("s") * NC + lax.axis_index("c")
        base = wid * cpw

        def body(i, carry):
            r = base + i
            pltpu.sync_copy(idx_hbm.at[r], idx_v)
            pltpu.async_copy(table_hbm.at[idx_v], rows_v, sem).wait()
            pltpu.sync_copy(rows_v, out_hbm.at[pl.ds(r * CHUNK, CHUNK)])
            return carry

        lax.fori_loop(0, cpw, body, 0)

    return k(table, idx2d)


def _ln_elu_body(x_ref, w_ref, b_ref, o_ref):
    x = x_ref[...]
    u = jnp.mean(x, axis=-1, keepdims=True)
    xc = x - u
    s = jnp.mean(xc * xc, axis=-1, keepdims=True)
    y = xc * lax.rsqrt(s + EPS)
    y = y * w_ref[...] + b_ref[...]
    o_ref[...] = jnp.where(y > 0, y, jnp.exp(jnp.minimum(y, 0.0)) - 1.0)


def _tc_ln_elu(x, w, b):
    N, D = x.shape
    R = 2048
    return pl.pallas_call(
        _ln_elu_body,
        grid=(N // R,),
        in_specs=[
            pl.BlockSpec((R, D), lambda i: (i, 0)),
            pl.BlockSpec((1, D), lambda i: (0, 0)),
            pl.BlockSpec((1, D), lambda i: (0, 0)),
        ],
        out_specs=pl.BlockSpec((R, D), lambda i: (i, 0)),
        out_shape=jax.ShapeDtypeStruct((N, D), jnp.float32),
    )(x, w.reshape(1, D), b.reshape(1, D))


def kernel(sequence, table, ln_weight, ln_bias):
    B, S = sequence.shape
    D = table.shape[1]
    idx2d = sequence.reshape(-1, CHUNK).astype(jnp.int32)
    g = _sc_gather(table, idx2d)
    out = _tc_ln_elu(g, ln_weight, ln_bias)
    return out.reshape(B, S, D)


# speedup vs baseline: 1.4738x; 1.4738x over previous
import functools

import jax
import jax.numpy as jnp
from jax import lax
from jax.experimental import pallas as pl
from jax.experimental.pallas import tpu as pltpu
from jax.experimental.pallas import tpu_sc as plsc

EPS = 1e-12
SUB = 128
SUBS_PER_CHUNK = 4
CHUNK = SUB * SUBS_PER_CHUNK


def _sc_gather(table2, idx2d, n_rows):
    n_idx_rows, _ = idx2d.shape
    D = table2.shape[1]
    info = plsc.get_sparse_core_info()
    NC, NS = info.num_cores, info.num_subcores
    NW = NC * NS
    rows_pw = n_idx_rows // NW
    cpw = rows_pw // SUBS_PER_CHUNK
    assert cpw % 2 == 0
    mesh = plsc.VectorSubcoreMesh(core_axis_name="c", subcore_axis_name="s")

    @functools.partial(
        pl.kernel,
        mesh=mesh,
        out_type=jax.ShapeDtypeStruct((n_rows, D), jnp.float32),
        scratch_types=[
            pltpu.VMEM((SUBS_PER_CHUNK, SUB), jnp.int32),
            pltpu.VMEM((SUBS_PER_CHUNK, SUB), jnp.int32),
            pltpu.VMEM((CHUNK, D), jnp.float32),
            pltpu.VMEM((CHUNK, D), jnp.float32),
            pltpu.SemaphoreType.DMA,
            pltpu.SemaphoreType.DMA,
        ],
        compiler_params=pltpu.CompilerParams(use_tc_tiling_on_sc=False),
    )
    def k(tab_hbm, idx_hbm, out_hbm, idx0, idx1, rows0, rows1, sem0, sem1):
        wid = lax.axis_index("s") * NC + lax.axis_index("c")
        base = wid * rows_pw

        def fire(c, idx_v, rows_v, sem):
            r0 = base + c * SUBS_PER_CHUNK
            pltpu.sync_copy(idx_hbm.at[pl.ds(r0, SUBS_PER_CHUNK)], idx_v)
            for j in range(SUBS_PER_CHUNK):
                pltpu.async_copy(
                    tab_hbm.at[idx_v.at[j]],
                    rows_v.at[pl.ds(j * SUB, SUB)],
                    sem,
                )

        def drain_write(c, idx_v, rows_v, sem):
            for j in range(SUBS_PER_CHUNK):
                pltpu.make_async_copy(
                    tab_hbm.at[idx_v.at[j]],
                    rows_v.at[pl.ds(j * SUB, SUB)],
                    sem,
                ).wait()
            flat = (base + c * SUBS_PER_CHUNK) * SUB
            pltpu.sync_copy(rows_v, out_hbm.at[pl.ds(flat, CHUNK)])

        def body(i, carry):
            c_even = 2 * i
            fire(c_even, idx0, rows0, sem0)

            @pl.when(i > 0)
            def _():
                drain_write(c_even - 1, idx1, rows1, sem1)

            fire(c_even + 1, idx1, rows1, sem1)
            drain_write(c_even, idx0, rows0, sem0)
            return carry

        lax.fori_loop(0, cpw // 2, body, 0)
        drain_write(cpw - 1, idx1, rows1, sem1)

    return k(table2, idx2d)


def _transpose_pack_body(x_ref, o_ref):
    D = x_ref.shape[0]
    y = x_ref[...].T
    y3 = y.reshape(y.shape[0] // 2, 2, D)
    o_ref[:, :D] = y3[:, 0, :]
    o_ref[:, D:] = y3[:, 1, :]


def _tc_transpose_pack(tT):
    D, V = tT.shape
    BN = 2048
    grid = (pl.cdiv(V, BN),)
    return pl.pallas_call(
        _transpose_pack_body,
        grid=grid,
        in_specs=[pl.BlockSpec((D, BN), lambda j: (0, j))],
        out_specs=pl.BlockSpec((BN // 2, 2 * D), lambda j: (j, 0)),
        out_shape=jax.ShapeDtypeStruct((V // 2, 2 * D), jnp.float32),
    )(tT)


def _ln_elu_body(x_ref, w_ref, b_ref, sel_ref, bc_ref, o_ref):
    x = x_ref[...]
    sel = sel_ref[...]
    bc = bc_ref[...]
    inv = 1.0 / 64.0
    sums = jax.lax.dot(x, sel, preferred_element_type=jnp.float32)
    u = jax.lax.dot(sums * inv, bc, preferred_element_type=jnp.float32)
    xc = x - u
    sq = jax.lax.dot(xc * xc, sel, preferred_element_type=jnp.float32)
    v = jax.lax.dot(sq * inv, bc, preferred_element_type=jnp.float32)
    y = xc * lax.rsqrt(v + EPS)
    y = y * w_ref[...] + b_ref[...]
    y = jnp.where(y > 0, y, jnp.exp(jnp.minimum(y, 0.0)) - 1.0)
    D = o_ref.shape[-1]
    o_ref[::2, :] = y[:, :D]
    o_ref[1::2, :] = y[:, D:]


def _tc_ln_elu(x2, w2, b2, sel, bc):
    N2, L = x2.shape
    R = 1024
    return pl.pallas_call(
        _ln_elu_body,
        grid=(N2 // R,),
        in_specs=[
            pl.BlockSpec((R, L), lambda i: (i, 0)),
            pl.BlockSpec((1, L), lambda i: (0, 0)),
            pl.BlockSpec((1, L), lambda i: (0, 0)),
            pl.BlockSpec((L, 2), lambda i: (0, 0)),
            pl.BlockSpec((2, L), lambda i: (0, 0)),
        ],
        out_specs=pl.BlockSpec((2 * R, L // 2), lambda i: (i, 0)),
        out_shape=jax.ShapeDtypeStruct((2 * N2, L // 2), jnp.float32),
    )(x2, w2, b2, sel, bc)


def kernel(sequence, table, ln_weight, ln_bias):
    B, S = sequence.shape
    V, D = table.shape
    n_rows = B * S
    tpack = _tc_transpose_pack(table.T)
    table2 = tpack.reshape(V, D)
    idx2d = sequence.astype(jnp.int32).reshape(-1, SUB)
    g = _sc_gather(table2, idx2d, n_rows)
    g2 = g.reshape(n_rows // 2, 2 * D)
    half = jnp.arange(2 * D, dtype=jnp.int32) >= D
    sel = jnp.stack([1.0 - half.astype(jnp.float32),
                     half.astype(jnp.float32)], axis=1)
    bc = sel.T
    w2 = jnp.concatenate([ln_weight, ln_weight]).reshape(1, 2 * D)
    b2 = jnp.concatenate([ln_bias, ln_bias]).reshape(1, 2 * D)
    out = _tc_ln_elu(g2, w2, b2, sel, bc)
    return out.reshape(B, S, D)
